# Initial kernel scaffold; baseline (speedup 1.0000x reference)
import jax
import jax.numpy as jnp
from jax.experimental import pallas as pl


def kernel(data):
    raise NotImplementedError("write your pallas kernel here")


# speedup vs baseline: 1.9137x; 1.9137x over previous
import jax
import jax.numpy as jnp
from jax.experimental import pallas as pl
from jax.experimental.pallas import tpu as pltpu

_K = 256
_SEED = 42
_NEG = -1e30


def _body(first_ref, dataT_vmem, g_ref, out_ref, closest, cvec, bval, bidx):
    k = pl.program_id(0)
    ci = pl.program_id(1)
    C = closest.shape[-1]

    @pl.when(ci == 0)
    def _new_round():
        idx = jnp.where(k == 0, first_ref[0], bidx[0])
        ck = idx // C
        cl = idx - ck * C
        chunk = dataT_vmem[ck]
        lanes = jax.lax.broadcasted_iota(jnp.int32, (1, C), 1)
        onehot = (lanes == cl).astype(jnp.float32)
        col = jnp.sum(chunk * onehot, axis=1, keepdims=True)
        cvec[...] = col
        out_ref[0] = col
        bval[0] = _NEG
        bidx[0] = 0

    c = cvec[...]
    x = dataT_vmem[ci]
    d = x - c
    newd = jnp.sum(d * d, axis=0, keepdims=True)
    cl_new = jnp.where(k == 0, newd, jnp.minimum(closest[ci], newd))
    closest[ci] = cl_new

    s = jnp.log(jnp.maximum(cl_new, 1e-12)) + g_ref[0, 0]
    m = jnp.max(s)
    iota = jax.lax.broadcasted_iota(jnp.int32, s.shape, 1)
    lidx = jnp.min(jnp.where(s == m, iota, C))
    gidx = lidx + ci * C
    better = m > bval[0]
    bval[0] = jnp.where(better, m, bval[0])
    bidx[0] = jnp.where(better, gidx, bidx[0])


def _kmeanspp(data, kk, nchunk, c, interpret=False):
    n, f = data.shape
    npad = nchunk * c
    key = jax.random.key(_SEED)
    first = jax.random.randint(
        jax.random.fold_in(key, 0), (), 0, n).astype(jnp.int32).reshape(1)
    keys = jax.vmap(lambda i: jax.random.fold_in(key, i))(jnp.arange(1, kk))
    g = jax.vmap(lambda kq: jax.random.gumbel(kq, (n,), jnp.float32))(keys)
    gp = jnp.full((kk + 1, npad), _NEG, jnp.float32).at[1:kk, :n].set(g)
    g4 = gp.reshape(kk + 1, nchunk, 1, c)
    dataTp = jnp.zeros((f, npad), jnp.float32).at[:, :n].set(data.T)
    dataT3 = dataTp.reshape(f, nchunk, c).transpose(1, 0, 2)

    grid_spec = pltpu.PrefetchScalarGridSpec(
        num_scalar_prefetch=1,
        grid=(kk, nchunk),
        in_specs=[
            pl.BlockSpec((nchunk, f, c), lambda k, ci, first: (0, 0, 0)),
            pl.BlockSpec((1, 1, 1, c), lambda k, ci, first: (k + 1, ci, 0, 0)),
        ],
        out_specs=pl.BlockSpec((1, f, 1), lambda k, ci, first: (k, 0, 0)),
        scratch_shapes=[
            pltpu.VMEM((nchunk, 1, c), jnp.float32),
            pltpu.VMEM((f, 1), jnp.float32),
            pltpu.SMEM((1,), jnp.float32),
            pltpu.SMEM((1,), jnp.int32),
        ],
    )
    out = pl.pallas_call(
        _body,
        grid_spec=grid_spec,
        out_shape=jax.ShapeDtypeStruct((kk, f, 1), jnp.float32),
        compiler_params=pltpu.CompilerParams(
            dimension_semantics=("arbitrary", "arbitrary")),
        interpret=interpret,
    )(first, dataT3, g4)
    return out.reshape(kk, f)


def kernel(data):
    return _kmeanspp(data, _K, 16, 6400)
